# baseline (speedup 1.0000x reference)
import functools
import jax
import jax.numpy as jnp
import numpy as np
from jax import lax
from jax.experimental import pallas as pl
from jax.experimental.pallas import tpu as pltpu
from jax.experimental.pallas import tpu_sc as plsc

N = 10000
E = 320000
D = 128
HEADS = 8
HID = 8
NC_OUT = 16

NPAD = 10240
EPAD = 327680
EPW = EPAD // 32
EC = 64
NCHUNK = EPW // EC
GC = 64
RPW = NPAD // 32

_mesh = plsc.VectorSubcoreMesh(core_axis_name="c", subcore_axis_name="s")


@functools.partial(
    pl.kernel,
    out_type=jax.ShapeDtypeStruct((NPAD, D), jnp.float32),
    mesh=_mesh,
    scratch_types=[
        pltpu.VMEM((GC,), jnp.int32),
        pltpu.VMEM((GC, D), jnp.float32),
        pltpu.SemaphoreType.DMA,
    ],
)
def _emb_gather(emb_hbm, idx_hbm, out_hbm, idxv, rows, sem):
    wid = lax.axis_index("s") * 2 + lax.axis_index("c")

    def chunk(ci, carry):
        base = wid * RPW + ci * GC
        pltpu.sync_copy(idx_hbm.at[pl.ds(base, GC)], idxv)
        pltpu.async_copy(emb_hbm.at[idxv], rows, sem).wait()
        pltpu.sync_copy(rows, out_hbm.at[pl.ds(base, GC)])
        return carry

    lax.fori_loop(0, RPW // GC, chunk, 0)


def _make_edge_kernel(edge_fn):
    @functools.partial(
        pl.kernel,
        out_type=jax.ShapeDtypeStruct((2, NPAD, 128), jnp.float32),
        mesh=_mesh,
        scratch_types=[
            pltpu.VMEM((2, EC), jnp.int32),
            pltpu.VMEM((2, EC), jnp.int32),
            pltpu.VMEM((2, EC, 128), jnp.float32),
            pltpu.VMEM((2, EC, 128), jnp.float32),
            pltpu.VMEM_SHARED((NPAD, 128), jnp.float32),
            pltpu.SemaphoreType.DMA,
            pltpu.SemaphoreType.DMA,
            pltpu.SemaphoreType.DMA,
            pltpu.SemaphoreType.DMA,
        ],
    )
    def k(src_hbm, dst_hbm, tsrc_hbm, acc_hbm,
          sidx, didx, srows, drows, accsh, ss0, ss1, sd0, sd1):
        cid = lax.axis_index("c")
        sid = lax.axis_index("s")
        sems_s = (ss0, ss1)
        sems_d = (sd0, sd1)
        epw0 = (EPAD // 4) // 16
        epw1 = (EPAD - EPAD // 4) // 16
        epw_c = jnp.where(cid == 0, epw0, epw1)
        off_c = jnp.where(cid == 0, 0, 16 * epw0)
        my_base = off_c + sid * epw_c
        nchunk_c = epw_c // EC

        zero16 = jnp.zeros((16,), jnp.float32)
        z = srows.at[0]

        def zrow(i, c):
            for kk in range(8):
                z[i, pl.ds(16 * kk, 16)] = zero16
            return c

        lax.fori_loop(0, EC, zrow, 0)
        rpw = NPAD // 16

        def zcp(kk, c):
            pltpu.sync_copy(z, accsh.at[pl.ds(sid * rpw + kk * EC, EC)])
            return c

        lax.fori_loop(0, rpw // EC, zcp, 0)
        plsc.subcore_barrier()

        def issue(g, b):
            base = my_base + g * EC
            pltpu.sync_copy(src_hbm.at[pl.ds(base, EC)], sidx.at[b])
            pltpu.sync_copy(dst_hbm.at[pl.ds(base, EC)], didx.at[b])
            pltpu.async_copy(tsrc_hbm.at[sidx.at[b]], srows.at[b], sems_s[b])
            pltpu.async_copy(tsrc_hbm.at[didx.at[b]], drows.at[b], sems_d[b])

        issue(0, 0)

        def outer(o, c):
            for b in range(2):
                g = 2 * o + b
                nb = 1 - b

                @pl.when(g + 1 < nchunk_c)
                def _():
                    issue(g + 1, nb)

                pltpu.make_async_copy(
                    tsrc_hbm.at[sidx.at[b]], srows.at[b], sems_s[b]).wait()
                pltpu.make_async_copy(
                    tsrc_hbm.at[didx.at[b]], drows.at[b], sems_d[b]).wait()
                edge_fn(srows.at[b], drows.at[b])
                pltpu.sync_copy(srows.at[b], accsh.at[didx.at[b]], add=True)
            return c

        lax.fori_loop(0, nchunk_c // 2, outer, 0)
        plsc.subcore_barrier()

        @pl.when(sid == 0)
        def _():
            pltpu.sync_copy(accsh, acc_hbm.at[cid])

    return k


def _edge1_fn(srows, drows):
    @plsc.parallel_loop(0, EC, 1, unroll=8)
    def edge(i):
        al = srows[i, pl.ds(64, 16)] + drows[i, pl.ds(80, 16)]
        al = jnp.where(al >= 0.0, al, 0.2 * al)
        e = jnp.exp(al)
        for k in range(4):
            srows[i, pl.ds(16 * k, 16)] = srows[i, pl.ds(16 * k, 16)] * e
        srows[i, pl.ds(64, 16)] = e


_edge1 = _make_edge_kernel(_edge1_fn)


def _edge2_fn(srows, drows):
    @plsc.parallel_loop(0, EC, 1, unroll=8)
    def edge(i):
        al = srows[i, pl.ds(16, 16)] + drows[i, pl.ds(32, 16)]
        al = jnp.where(al >= 0.0, al, 0.2 * al)
        e = jnp.exp(al)
        srows[i, pl.ds(0, 16)] = srows[i, pl.ds(0, 16)] * e
        srows[i, pl.ds(16, 16)] = e


_edge2 = _make_edge_kernel(_edge2_fn)


def _tab1_body(h0_ref, w1p_ref, a2s_ref, a2d_ref, tsrc_ref):
    h1p = jnp.dot(h0_ref[...], w1p_ref[...], preferred_element_type=jnp.float32)
    asrc2 = jnp.dot(h1p, a2s_ref[...], preferred_element_type=jnp.float32)
    adst2 = jnp.dot(h1p, a2d_ref[...], preferred_element_type=jnp.float32)
    zeros32 = jnp.zeros((h1p.shape[0], 32), jnp.float32)
    tsrc_ref[...] = jnp.concatenate([h1p, asrc2, adst2, zeros32], axis=1)


def _tab1(h0, W1p, A2s, A2d):
    bm = 512
    return pl.pallas_call(
        _tab1_body,
        out_shape=jax.ShapeDtypeStruct((NPAD, 128), jnp.float32),
        grid=(NPAD // bm,),
        in_specs=[
            pl.BlockSpec((bm, D), lambda i: (i, 0)),
            pl.BlockSpec((D, 64), lambda i: (0, 0)),
            pl.BlockSpec((64, 16), lambda i: (0, 0)),
            pl.BlockSpec((64, 16), lambda i: (0, 0)),
        ],
        out_specs=pl.BlockSpec((bm, 128), lambda i: (i, 0)),
    )(h0, W1p, A2s, A2d)


def _tab2_body(a0_ref, a1_ref, r8p_ref, pmt_ref, b1_ref, w2_ref,
               a2s_ref, a2d_ref, t2s_ref):
    acc = a0_ref[...] + a1_ref[...]
    msgp = acc[:, :64]
    den = acc[:, 64:72]
    denp = jnp.dot(den, r8p_ref[...], preferred_element_type=jnp.float32)
    out1p = msgp / (denp + 1e-16)
    out1 = jnp.dot(out1p, pmt_ref[...], preferred_element_type=jnp.float32)
    z = out1 + b1_ref[0:1, :]
    h2 = jnp.where(z > 0.0, z, jnp.exp(z) - 1.0)
    g2 = jnp.dot(h2, w2_ref[...], preferred_element_type=jnp.float32)
    s2 = jnp.dot(g2, a2s_ref[...], preferred_element_type=jnp.float32)
    d2 = jnp.dot(g2, a2d_ref[...], preferred_element_type=jnp.float32)
    zeros80 = jnp.zeros((g2.shape[0], 80), jnp.float32)
    t2s_ref[...] = jnp.concatenate([g2, s2, d2, zeros80], axis=1)


def _tab2(acc0, acc1, R8p, PmT, b1m, W2, A2s2, A2d2):
    bm = 512
    return pl.pallas_call(
        _tab2_body,
        out_shape=jax.ShapeDtypeStruct((NPAD, 128), jnp.float32),
        grid=(NPAD // bm,),
        in_specs=[
            pl.BlockSpec((bm, 128), lambda i: (i, 0)),
            pl.BlockSpec((bm, 128), lambda i: (i, 0)),
            pl.BlockSpec((8, 64), lambda i: (0, 0)),
            pl.BlockSpec((64, 64), lambda i: (0, 0)),
            pl.BlockSpec((8, 64), lambda i: (0, 0)),
            pl.BlockSpec((64, 16), lambda i: (0, 0)),
            pl.BlockSpec((16, 16), lambda i: (0, 0)),
            pl.BlockSpec((16, 16), lambda i: (0, 0)),
        ],
        out_specs=pl.BlockSpec((bm, 128), lambda i: (i, 0)),
    )(acc0, acc1, R8p, PmT, b1m, W2, A2s2, A2d2)


def _fin_body(a0_ref, a1_ref, b2_ref, out_ref):
    acc = a0_ref[...] + a1_ref[...]
    msg = acc[:, :NC_OUT]
    den = acc[:, NC_OUT:2 * NC_OUT]
    out = msg / (den + 1e-16) + b2_ref[0:1, :]
    m = jnp.max(out, axis=-1, keepdims=True)
    s = out - m
    lse = jnp.log(jnp.sum(jnp.exp(s), axis=-1, keepdims=True))
    out_ref[...] = s - lse


def _finalize(acc0, acc1, b2m):
    bm = 512
    return pl.pallas_call(
        _fin_body,
        out_shape=jax.ShapeDtypeStruct((NPAD, NC_OUT), jnp.float32),
        grid=(NPAD // bm,),
        in_specs=[
            pl.BlockSpec((bm, 128), lambda i: (i, 0)),
            pl.BlockSpec((bm, 128), lambda i: (i, 0)),
            pl.BlockSpec((8, NC_OUT), lambda i: (0, 0)),
        ],
        out_specs=pl.BlockSpec((bm, NC_OUT), lambda i: (i, 0)),
    )(acc0, acc1, b2m)


def kernel(x, edge_index, emb, W1, a_src1, a_dst1, b1, W2, a_src2, a_dst2, b2):
    f32 = jnp.float32

    j = np.arange(64)
    prime_of_orig = (j % 8) * 8 + (j // 8)
    Pm = np.zeros((64, 64), np.float32)
    Pm[j, prime_of_orig] = 1.0
    Pm = jnp.asarray(Pm)
    W1p = W1 @ Pm
    rows = (np.arange(64) % 8) * 8 + (np.arange(64) // 8)
    h_idx = np.arange(64) // 8
    k_idx = np.arange(64) % 8
    Aps = jnp.zeros((64, 8), f32).at[rows, h_idx].set(a_src1[h_idx, k_idx])
    Apd = jnp.zeros((64, 8), f32).at[rows, h_idx].set(a_dst1[h_idx, k_idx])
    A2s = jnp.concatenate([Aps, Aps], axis=1)
    A2d = jnp.concatenate([Apd, Apd], axis=1)
    R8p = jnp.zeros((8, 64), f32).at[h_idx, rows].set(1.0)
    PmT = Pm.T
    b1m = jnp.broadcast_to(b1[None, :], (8, 64))
    A2s2 = jnp.broadcast_to(a_src2[0][:, None], (16, 16))
    A2d2 = jnp.broadcast_to(a_dst2[0][:, None], (16, 16))
    b2m = jnp.broadcast_to(b2[None, :], (8, NC_OUT))

    idx = jnp.pad(x[:, 0].astype(jnp.int32), (0, NPAD - N))
    srcp = jnp.pad(edge_index[0].astype(jnp.int32), (0, EPAD - E),
                   constant_values=N)
    dstp = jnp.pad(edge_index[1].astype(jnp.int32), (0, EPAD - E),
                   constant_values=N)
    h0 = _emb_gather(emb, idx)
    tsrc = _tab1(h0, W1p, A2s, A2d)
    acc1 = _edge1(srcp, dstp, tsrc)
    t2s = _tab2(acc1[0], acc1[1], R8p, PmT, b1m, W2, A2s2, A2d2)
    acc2 = _edge2(srcp, dstp, t2s)
    out = _finalize(acc2[0], acc2[1], b2m)
    return out[:N]


# speedup vs baseline: 1.1595x; 1.1595x over previous
import functools
import jax
import jax.numpy as jnp
import numpy as np
from jax import lax
from jax.experimental import pallas as pl
from jax.experimental.pallas import tpu as pltpu
from jax.experimental.pallas import tpu_sc as plsc

N = 10000
E = 320000
D = 128
HEADS = 8
HID = 8
NC_OUT = 16

NPAD = 10240
EPAD = 327680
EPW = EPAD // 32
EC = 64
NCHUNK = EPW // EC
GC = 64
RPW = NPAD // 32

_mesh = plsc.VectorSubcoreMesh(core_axis_name="c", subcore_axis_name="s")


@functools.partial(
    pl.kernel,
    out_type=jax.ShapeDtypeStruct((NPAD, D), jnp.float32),
    mesh=_mesh,
    scratch_types=[
        pltpu.VMEM((GC,), jnp.int32),
        pltpu.VMEM((GC, D), jnp.float32),
        pltpu.SemaphoreType.DMA,
    ],
)
def _emb_gather(emb_hbm, idx_hbm, out_hbm, idxv, rows, sem):
    wid = lax.axis_index("s") * 2 + lax.axis_index("c")

    def chunk(ci, carry):
        base = wid * RPW + ci * GC
        pltpu.sync_copy(idx_hbm.at[pl.ds(base, GC)], idxv)
        pltpu.async_copy(emb_hbm.at[idxv], rows, sem).wait()
        pltpu.sync_copy(rows, out_hbm.at[pl.ds(base, GC)])
        return carry

    lax.fori_loop(0, RPW // GC, chunk, 0)


def _make_edge_kernel(edge_fn):
    @functools.partial(
        pl.kernel,
        out_type=jax.ShapeDtypeStruct((2, NPAD, 128), jnp.float32),
        mesh=_mesh,
        scratch_types=[
            pltpu.VMEM((2, EC), jnp.int32),
            pltpu.VMEM((2, EC), jnp.int32),
            pltpu.VMEM((2, EC, 128), jnp.float32),
            pltpu.VMEM((2, EC, 128), jnp.float32),
            pltpu.VMEM_SHARED((NPAD, 128), jnp.float32),
            pltpu.SemaphoreType.DMA,
            pltpu.SemaphoreType.DMA,
            pltpu.SemaphoreType.DMA,
            pltpu.SemaphoreType.DMA,
        ],
    )
    def k(src_hbm, dst_hbm, tsrc_hbm, acc_hbm,
          sidx, didx, srows, drows, accsh, ss0, ss1, sd0, sd1):
        cid = lax.axis_index("c")
        sid = lax.axis_index("s")
        sems_s = (ss0, ss1)
        sems_d = (sd0, sd1)
        epw0 = (EPAD - EPAD // 4) // 16
        epw1 = (EPAD // 4) // 16
        epw_c = jnp.where(cid == 0, epw0, epw1)
        off_c = jnp.where(cid == 0, 0, 16 * epw0)
        my_base = off_c + sid * epw_c
        nchunk_c = epw_c // EC

        zero16 = jnp.zeros((16,), jnp.float32)
        z = srows.at[0]

        def zrow(i, c):
            for kk in range(8):
                z[i, pl.ds(16 * kk, 16)] = zero16
            return c

        lax.fori_loop(0, EC, zrow, 0)
        rpw = NPAD // 16

        def zcp(kk, c):
            pltpu.sync_copy(z, accsh.at[pl.ds(sid * rpw + kk * EC, EC)])
            return c

        lax.fori_loop(0, rpw // EC, zcp, 0)
        plsc.subcore_barrier()

        def issue(g, b):
            base = my_base + g * EC
            pltpu.sync_copy(src_hbm.at[pl.ds(base, EC)], sidx.at[b])
            pltpu.sync_copy(dst_hbm.at[pl.ds(base, EC)], didx.at[b])
            pltpu.async_copy(tsrc_hbm.at[sidx.at[b]], srows.at[b], sems_s[b])
            pltpu.async_copy(tsrc_hbm.at[didx.at[b]], drows.at[b], sems_d[b])

        issue(0, 0)

        def outer(o, c):
            for b in range(2):
                g = 2 * o + b
                nb = 1 - b

                @pl.when(g + 1 < nchunk_c)
                def _():
                    issue(g + 1, nb)

                pltpu.make_async_copy(
                    tsrc_hbm.at[sidx.at[b]], srows.at[b], sems_s[b]).wait()
                pltpu.make_async_copy(
                    tsrc_hbm.at[didx.at[b]], drows.at[b], sems_d[b]).wait()
                edge_fn(srows.at[b], drows.at[b])
                pltpu.sync_copy(srows.at[b], accsh.at[didx.at[b]], add=True)
            return c

        lax.fori_loop(0, nchunk_c // 2, outer, 0)
        plsc.subcore_barrier()

        @pl.when(sid == 0)
        def _():
            pltpu.sync_copy(accsh, acc_hbm.at[cid])

    return k


def _edge1_fn(srows, drows):
    @plsc.parallel_loop(0, EC, 1, unroll=8)
    def edge(i):
        al = srows[i, pl.ds(64, 16)] + drows[i, pl.ds(80, 16)]
        al = jnp.where(al >= 0.0, al, 0.2 * al)
        e = jnp.exp(al)
        for k in range(4):
            srows[i, pl.ds(16 * k, 16)] = srows[i, pl.ds(16 * k, 16)] * e
        srows[i, pl.ds(64, 16)] = e


_edge1 = _make_edge_kernel(_edge1_fn)


def _edge2_fn(srows, drows):
    @plsc.parallel_loop(0, EC, 1, unroll=8)
    def edge(i):
        al = srows[i, pl.ds(16, 16)] + drows[i, pl.ds(32, 16)]
        al = jnp.where(al >= 0.0, al, 0.2 * al)
        e = jnp.exp(al)
        srows[i, pl.ds(0, 16)] = srows[i, pl.ds(0, 16)] * e
        srows[i, pl.ds(16, 16)] = e


_edge2 = _make_edge_kernel(_edge2_fn)


def _tab1_body(h0_ref, w1p_ref, a2s_ref, a2d_ref, tsrc_ref):
    h1p = jnp.dot(h0_ref[...], w1p_ref[...], preferred_element_type=jnp.float32)
    asrc2 = jnp.dot(h1p, a2s_ref[...], preferred_element_type=jnp.float32)
    adst2 = jnp.dot(h1p, a2d_ref[...], preferred_element_type=jnp.float32)
    zeros32 = jnp.zeros((h1p.shape[0], 32), jnp.float32)
    tsrc_ref[...] = jnp.concatenate([h1p, asrc2, adst2, zeros32], axis=1)


def _tab1(h0, W1p, A2s, A2d):
    bm = 512
    return pl.pallas_call(
        _tab1_body,
        out_shape=jax.ShapeDtypeStruct((NPAD, 128), jnp.float32),
        grid=(NPAD // bm,),
        in_specs=[
            pl.BlockSpec((bm, D), lambda i: (i, 0)),
            pl.BlockSpec((D, 64), lambda i: (0, 0)),
            pl.BlockSpec((64, 16), lambda i: (0, 0)),
            pl.BlockSpec((64, 16), lambda i: (0, 0)),
        ],
        out_specs=pl.BlockSpec((bm, 128), lambda i: (i, 0)),
    )(h0, W1p, A2s, A2d)


def _tab2_body(a0_ref, a1_ref, r8p_ref, pmt_ref, b1_ref, w2_ref,
               a2s_ref, a2d_ref, t2s_ref):
    acc = a0_ref[...] + a1_ref[...]
    msgp = acc[:, :64]
    den = acc[:, 64:72]
    denp = jnp.dot(den, r8p_ref[...], preferred_element_type=jnp.float32)
    out1p = msgp / (denp + 1e-16)
    out1 = jnp.dot(out1p, pmt_ref[...], preferred_element_type=jnp.float32)
    z = out1 + b1_ref[0:1, :]
    h2 = jnp.where(z > 0.0, z, jnp.exp(z) - 1.0)
    g2 = jnp.dot(h2, w2_ref[...], preferred_element_type=jnp.float32)
    s2 = jnp.dot(g2, a2s_ref[...], preferred_element_type=jnp.float32)
    d2 = jnp.dot(g2, a2d_ref[...], preferred_element_type=jnp.float32)
    zeros80 = jnp.zeros((g2.shape[0], 80), jnp.float32)
    t2s_ref[...] = jnp.concatenate([g2, s2, d2, zeros80], axis=1)


def _tab2(acc0, acc1, R8p, PmT, b1m, W2, A2s2, A2d2):
    bm = 512
    return pl.pallas_call(
        _tab2_body,
        out_shape=jax.ShapeDtypeStruct((NPAD, 128), jnp.float32),
        grid=(NPAD // bm,),
        in_specs=[
            pl.BlockSpec((bm, 128), lambda i: (i, 0)),
            pl.BlockSpec((bm, 128), lambda i: (i, 0)),
            pl.BlockSpec((8, 64), lambda i: (0, 0)),
            pl.BlockSpec((64, 64), lambda i: (0, 0)),
            pl.BlockSpec((8, 64), lambda i: (0, 0)),
            pl.BlockSpec((64, 16), lambda i: (0, 0)),
            pl.BlockSpec((16, 16), lambda i: (0, 0)),
            pl.BlockSpec((16, 16), lambda i: (0, 0)),
        ],
        out_specs=pl.BlockSpec((bm, 128), lambda i: (i, 0)),
    )(acc0, acc1, R8p, PmT, b1m, W2, A2s2, A2d2)


def _fin_body(a0_ref, a1_ref, b2_ref, out_ref):
    acc = a0_ref[...] + a1_ref[...]
    msg = acc[:, :NC_OUT]
    den = acc[:, NC_OUT:2 * NC_OUT]
    out = msg / (den + 1e-16) + b2_ref[0:1, :]
    m = jnp.max(out, axis=-1, keepdims=True)
    s = out - m
    lse = jnp.log(jnp.sum(jnp.exp(s), axis=-1, keepdims=True))
    out_ref[...] = s - lse


def _finalize(acc0, acc1, b2m):
    bm = 512
    return pl.pallas_call(
        _fin_body,
        out_shape=jax.ShapeDtypeStruct((NPAD, NC_OUT), jnp.float32),
        grid=(NPAD // bm,),
        in_specs=[
            pl.BlockSpec((bm, 128), lambda i: (i, 0)),
            pl.BlockSpec((bm, 128), lambda i: (i, 0)),
            pl.BlockSpec((8, NC_OUT), lambda i: (0, 0)),
        ],
        out_specs=pl.BlockSpec((bm, NC_OUT), lambda i: (i, 0)),
    )(acc0, acc1, b2m)


def kernel(x, edge_index, emb, W1, a_src1, a_dst1, b1, W2, a_src2, a_dst2, b2):
    f32 = jnp.float32

    j = np.arange(64)
    prime_of_orig = (j % 8) * 8 + (j // 8)
    Pm = np.zeros((64, 64), np.float32)
    Pm[j, prime_of_orig] = 1.0
    Pm = jnp.asarray(Pm)
    W1p = W1 @ Pm
    rows = (np.arange(64) % 8) * 8 + (np.arange(64) // 8)
    h_idx = np.arange(64) // 8
    k_idx = np.arange(64) % 8
    Aps = jnp.zeros((64, 8), f32).at[rows, h_idx].set(a_src1[h_idx, k_idx])
    Apd = jnp.zeros((64, 8), f32).at[rows, h_idx].set(a_dst1[h_idx, k_idx])
    A2s = jnp.concatenate([Aps, Aps], axis=1)
    A2d = jnp.concatenate([Apd, Apd], axis=1)
    R8p = jnp.zeros((8, 64), f32).at[h_idx, rows].set(1.0)
    PmT = Pm.T
    b1m = jnp.broadcast_to(b1[None, :], (8, 64))
    A2s2 = jnp.broadcast_to(a_src2[0][:, None], (16, 16))
    A2d2 = jnp.broadcast_to(a_dst2[0][:, None], (16, 16))
    b2m = jnp.broadcast_to(b2[None, :], (8, NC_OUT))

    idx = jnp.pad(x[:, 0].astype(jnp.int32), (0, NPAD - N))
    srcp = jnp.pad(edge_index[0].astype(jnp.int32), (0, EPAD - E),
                   constant_values=N)
    dstp = jnp.pad(edge_index[1].astype(jnp.int32), (0, EPAD - E),
                   constant_values=N)
    h0 = _emb_gather(emb, idx)
    tsrc = _tab1(h0, W1p, A2s, A2d)
    acc1 = _edge1(srcp, dstp, tsrc)
    t2s = _tab2(acc1[0], acc1[1], R8p, PmT, b1m, W2, A2s2, A2d2)
    acc2 = _edge2(srcp, dstp, t2s)
    out = _finalize(acc2[0], acc2[1], b2m)
    return out[:N]


# speedup vs baseline: 1.6448x; 1.4185x over previous
import functools
import jax
import jax.numpy as jnp
import numpy as np
from jax import lax
from jax.experimental import pallas as pl
from jax.experimental.pallas import tpu as pltpu
from jax.experimental.pallas import tpu_sc as plsc

N = 10000
E = 320000
D = 128
HEADS = 8
HID = 8
NC_OUT = 16

NPAD = 10240
EPAD = 327680
EPW = EPAD // 32
EC = 64
NCHUNK = EPW // EC
GC = 64
RPW = NPAD // 32

_mesh = plsc.VectorSubcoreMesh(core_axis_name="c", subcore_axis_name="s")


@functools.partial(
    pl.kernel,
    out_type=jax.ShapeDtypeStruct((NPAD, D), jnp.float32),
    mesh=_mesh,
    scratch_types=[
        pltpu.VMEM((GC,), jnp.int32),
        pltpu.VMEM((GC, D), jnp.float32),
        pltpu.SemaphoreType.DMA,
    ],
)
def _emb_gather(emb_hbm, idx_hbm, out_hbm, idxv, rows, sem):
    wid = lax.axis_index("s") * 2 + lax.axis_index("c")

    def chunk(ci, carry):
        base = wid * RPW + ci * GC
        pltpu.sync_copy(idx_hbm.at[pl.ds(base, GC)], idxv)
        pltpu.async_copy(emb_hbm.at[idxv], rows, sem).wait()
        pltpu.sync_copy(rows, out_hbm.at[pl.ds(base, GC)])
        return carry

    lax.fori_loop(0, RPW // GC, chunk, 0)


def _make_edge_kernel(edge_fn):
    @functools.partial(
        pl.kernel,
        out_type=jax.ShapeDtypeStruct((2, NPAD, 128), jnp.float32),
        mesh=_mesh,
        compiler_params=pltpu.CompilerParams(use_tc_tiling_on_sc=False),
        scratch_types=[
            pltpu.VMEM((2, EC), jnp.int32),
            pltpu.VMEM((2, EC), jnp.int32),
            pltpu.VMEM((2, EC, 64), jnp.int32),
            pltpu.VMEM((2, EC, 64), jnp.int32),
            pltpu.VMEM((EC, 128), jnp.float32),
            pltpu.VMEM_SHARED((NPAD, 128), jnp.float32),
            pltpu.SemaphoreType.DMA,
            pltpu.SemaphoreType.DMA,
            pltpu.SemaphoreType.DMA,
            pltpu.SemaphoreType.DMA,
        ],
    )
    def k(src_hbm, dst_hbm, tsrc_hbm, acc_hbm,
          sidx, didx, srows, drows, orows, accsh, ss0, ss1, sd0, sd1):
        cid = lax.axis_index("c")
        sid = lax.axis_index("s")
        sems_s = (ss0, ss1)
        sems_d = (sd0, sd1)
        epw0 = (EPAD - EPAD // 4) // 16
        epw1 = (EPAD // 4) // 16
        epw_c = jnp.where(cid == 0, epw0, epw1)
        off_c = jnp.where(cid == 0, 0, 16 * epw0)
        my_base = off_c + sid * epw_c
        nchunk_c = epw_c // EC

        zero16 = jnp.zeros((16,), jnp.float32)

        def zrow(i, c):
            for kk in range(8):
                orows[i, pl.ds(16 * kk, 16)] = zero16
            return c

        lax.fori_loop(0, EC, zrow, 0)
        rpw = NPAD // 16

        def zcp(kk, c):
            pltpu.sync_copy(orows, accsh.at[pl.ds(sid * rpw + kk * EC, EC)])
            return c

        lax.fori_loop(0, rpw // EC, zcp, 0)
        plsc.subcore_barrier()

        def issue(g, b):
            base = my_base + g * EC
            pltpu.sync_copy(src_hbm.at[pl.ds(base, EC)], sidx.at[b])
            pltpu.sync_copy(dst_hbm.at[pl.ds(base, EC)], didx.at[b])
            pltpu.async_copy(tsrc_hbm.at[sidx.at[b]], srows.at[b], sems_s[b])
            pltpu.async_copy(tsrc_hbm.at[didx.at[b]], drows.at[b], sems_d[b])

        issue(0, 0)

        def outer(o, c):
            for b in range(2):
                g = 2 * o + b
                nb = 1 - b

                @pl.when(g + 1 < nchunk_c)
                def _():
                    issue(g + 1, nb)

                pltpu.make_async_copy(
                    tsrc_hbm.at[sidx.at[b]], srows.at[b], sems_s[b]).wait()
                pltpu.make_async_copy(
                    tsrc_hbm.at[didx.at[b]], drows.at[b], sems_d[b]).wait()
                edge_fn(srows.at[b], drows.at[b], orows)
                pltpu.sync_copy(orows, accsh.at[didx.at[b]], add=True)
            return c

        lax.fori_loop(0, nchunk_c // 2, outer, 0)
        plsc.subcore_barrier()

        @pl.when(sid == 0)
        def _():
            pltpu.sync_copy(accsh, acc_hbm.at[cid])

    return k


def _lo16(v):
    return jax.lax.bitcast_convert_type(jnp.left_shift(v, 16), jnp.float32)


def _hi16(v):
    mask = jnp.full((16,), -65536, jnp.int32)
    return jax.lax.bitcast_convert_type(jnp.bitwise_and(v, mask), jnp.float32)


def _edge1_fn(srows, drows, orows):
    for i in range(EC):
        asrc = _lo16(srows[i, pl.ds(32, 16)])
        adst = _hi16(drows[i, pl.ds(32, 16)])
        al = asrc + adst
        al = jnp.where(al >= 0.0, al, 0.2 * al)
        e = jnp.exp(al)
        ha = srows[i, pl.ds(0, 16)]
        hb = srows[i, pl.ds(16, 16)]
        orows[i, pl.ds(0, 16)] = _lo16(ha) * e
        orows[i, pl.ds(16, 16)] = _hi16(ha) * e
        orows[i, pl.ds(32, 16)] = _lo16(hb) * e
        orows[i, pl.ds(48, 16)] = _hi16(hb) * e
        orows[i, pl.ds(64, 16)] = e


_edge1 = _make_edge_kernel(_edge1_fn)


def _edge2_fn(srows, drows, orows):
    for i in range(EC):
        s0 = srows[i, pl.ds(0, 16)]
        asrc = _hi16(s0)
        adst = _lo16(drows[i, pl.ds(16, 16)])
        al = asrc + adst
        al = jnp.where(al >= 0.0, al, 0.2 * al)
        e = jnp.exp(al)
        orows[i, pl.ds(0, 16)] = _lo16(s0) * e
        orows[i, pl.ds(16, 16)] = e


_edge2 = _make_edge_kernel(_edge2_fn)


def _rne16(x):
    xb = jax.lax.bitcast_convert_type(x, jnp.int32)
    r = jnp.right_shift(xb + 0x7FFF + jnp.bitwise_and(
        jnp.right_shift(xb, 16), 1), 16)
    return jnp.bitwise_and(r, 0xFFFF)


def _pack16(a, b):
    return jnp.bitwise_or(_rne16(a), jnp.left_shift(_rne16(b), 16))


def _tab1_body(h0_ref, w1p_ref, a2s_ref, a2d_ref, tsrc_ref):
    h1p = jnp.dot(h0_ref[...], w1p_ref[...], preferred_element_type=jnp.float32)
    asrc2 = jnp.dot(h1p, a2s_ref[...], preferred_element_type=jnp.float32)
    adst2 = jnp.dot(h1p, a2d_ref[...], preferred_element_type=jnp.float32)
    zeros16 = jnp.zeros((h1p.shape[0], 16), jnp.float32)
    a = jnp.concatenate([h1p[:, 0:16], h1p[:, 32:48], asrc2[:, 0:16],
                         zeros16], axis=1)
    b = jnp.concatenate([h1p[:, 16:32], h1p[:, 48:64], adst2[:, 0:16],
                         zeros16], axis=1)
    tsrc_ref[...] = _pack16(a, b)


def _tab1(h0, W1p, A2s, A2d):
    bm = 512
    return pl.pallas_call(
        _tab1_body,
        out_shape=jax.ShapeDtypeStruct((NPAD, 64), jnp.int32),
        grid=(NPAD // bm,),
        in_specs=[
            pl.BlockSpec((bm, D), lambda i: (i, 0)),
            pl.BlockSpec((D, 64), lambda i: (0, 0)),
            pl.BlockSpec((64, 16), lambda i: (0, 0)),
            pl.BlockSpec((64, 16), lambda i: (0, 0)),
        ],
        out_specs=pl.BlockSpec((bm, 64), lambda i: (i, 0)),
    )(h0, W1p, A2s, A2d)


def _tab2_body(a0_ref, a1_ref, r8p_ref, pmt_ref, b1_ref, w2_ref,
               a2s_ref, a2d_ref, t2s_ref):
    acc = a0_ref[...] + a1_ref[...]
    msgp = acc[:, :64]
    den = acc[:, 64:72]
    denp = jnp.dot(den, r8p_ref[...], preferred_element_type=jnp.float32)
    out1p = msgp / (denp + 1e-16)
    out1 = jnp.dot(out1p, pmt_ref[...], preferred_element_type=jnp.float32)
    z = out1 + b1_ref[0:1, :]
    h2 = jnp.where(z > 0.0, z, jnp.exp(z) - 1.0)
    g2 = jnp.dot(h2, w2_ref[...], preferred_element_type=jnp.float32)
    s2 = jnp.dot(g2, a2s_ref[...], preferred_element_type=jnp.float32)
    d2 = jnp.dot(g2, a2d_ref[...], preferred_element_type=jnp.float32)
    zeros16 = jnp.zeros((g2.shape[0], 16), jnp.float32)
    a = jnp.concatenate([g2, d2, zeros16, zeros16], axis=1)
    b = jnp.concatenate([s2, zeros16, zeros16, zeros16], axis=1)
    t2s_ref[...] = _pack16(a, b)


def _tab2(acc0, acc1, R8p, PmT, b1m, W2, A2s2, A2d2):
    bm = 512
    return pl.pallas_call(
        _tab2_body,
        out_shape=jax.ShapeDtypeStruct((NPAD, 64), jnp.int32),
        grid=(NPAD // bm,),
        in_specs=[
            pl.BlockSpec((bm, 128), lambda i: (i, 0)),
            pl.BlockSpec((bm, 128), lambda i: (i, 0)),
            pl.BlockSpec((8, 64), lambda i: (0, 0)),
            pl.BlockSpec((64, 64), lambda i: (0, 0)),
            pl.BlockSpec((8, 64), lambda i: (0, 0)),
            pl.BlockSpec((64, 16), lambda i: (0, 0)),
            pl.BlockSpec((16, 16), lambda i: (0, 0)),
            pl.BlockSpec((16, 16), lambda i: (0, 0)),
        ],
        out_specs=pl.BlockSpec((bm, 64), lambda i: (i, 0)),
    )(acc0, acc1, R8p, PmT, b1m, W2, A2s2, A2d2)


def _fin_body(a0_ref, a1_ref, b2_ref, out_ref):
    acc = a0_ref[...] + a1_ref[...]
    msg = acc[:, :NC_OUT]
    den = acc[:, NC_OUT:2 * NC_OUT]
    out = msg / (den + 1e-16) + b2_ref[0:1, :]
    m = jnp.max(out, axis=-1, keepdims=True)
    s = out - m
    lse = jnp.log(jnp.sum(jnp.exp(s), axis=-1, keepdims=True))
    out_ref[...] = s - lse


def _finalize(acc0, acc1, b2m):
    bm = 512
    return pl.pallas_call(
        _fin_body,
        out_shape=jax.ShapeDtypeStruct((NPAD, NC_OUT), jnp.float32),
        grid=(NPAD // bm,),
        in_specs=[
            pl.BlockSpec((bm, 128), lambda i: (i, 0)),
            pl.BlockSpec((bm, 128), lambda i: (i, 0)),
            pl.BlockSpec((8, NC_OUT), lambda i: (0, 0)),
        ],
        out_specs=pl.BlockSpec((bm, NC_OUT), lambda i: (i, 0)),
    )(acc0, acc1, b2m)


def kernel(x, edge_index, emb, W1, a_src1, a_dst1, b1, W2, a_src2, a_dst2, b2):
    f32 = jnp.float32

    j = np.arange(64)
    prime_of_orig = (j % 8) * 8 + (j // 8)
    Pm = np.zeros((64, 64), np.float32)
    Pm[j, prime_of_orig] = 1.0
    Pm = jnp.asarray(Pm)
    W1p = W1 @ Pm
    rows = (np.arange(64) % 8) * 8 + (np.arange(64) // 8)
    h_idx = np.arange(64) // 8
    k_idx = np.arange(64) % 8
    Aps = jnp.zeros((64, 8), f32).at[rows, h_idx].set(a_src1[h_idx, k_idx])
    Apd = jnp.zeros((64, 8), f32).at[rows, h_idx].set(a_dst1[h_idx, k_idx])
    A2s = jnp.concatenate([Aps, Aps], axis=1)
    A2d = jnp.concatenate([Apd, Apd], axis=1)
    R8p = jnp.zeros((8, 64), f32).at[h_idx, rows].set(1.0)
    PmT = Pm.T
    b1m = jnp.broadcast_to(b1[None, :], (8, 64))
    A2s2 = jnp.broadcast_to(a_src2[0][:, None], (16, 16))
    A2d2 = jnp.broadcast_to(a_dst2[0][:, None], (16, 16))
    b2m = jnp.broadcast_to(b2[None, :], (8, NC_OUT))


    idx = jnp.pad(x[:, 0].astype(jnp.int32), (0, NPAD - N))
    srcp = jnp.pad(edge_index[0].astype(jnp.int32), (0, EPAD - E),
                   constant_values=N)
    dstp = jnp.pad(edge_index[1].astype(jnp.int32), (0, EPAD - E),
                   constant_values=N)
    h0 = _emb_gather(emb, idx)
    tsrc = _tab1(h0, W1p, A2s, A2d)
    acc1 = _edge1(srcp, dstp, tsrc)
    t2s = _tab2(acc1[0], acc1[1], R8p, PmT, b1m, W2, A2s2, A2d2)
    acc2 = _edge2(srcp, dstp, t2s)
    out = _finalize(acc2[0], acc2[1], b2m)
    return out[:N]


# speedup vs baseline: 2.4950x; 1.5169x over previous
import functools
import jax
import jax.numpy as jnp
import numpy as np
from jax import lax
from jax.experimental import pallas as pl
from jax.experimental.pallas import tpu as pltpu
from jax.experimental.pallas import tpu_sc as plsc

N = 10000
E = 320000
D = 128
HEADS = 8
HID = 8
NC_OUT = 16

NPAD = 10240
EPAD = 327680
EPW = EPAD // 32
EC = 64
NCHUNK = EPW // EC
GC = 64
RPW = NPAD // 32

_mesh = plsc.VectorSubcoreMesh(core_axis_name="c", subcore_axis_name="s")


@functools.partial(
    pl.kernel,
    out_type=jax.ShapeDtypeStruct((NPAD, D), jnp.float32),
    mesh=_mesh,
    scratch_types=[
        pltpu.VMEM((GC,), jnp.int32),
        pltpu.VMEM((GC, D), jnp.float32),
        pltpu.SemaphoreType.DMA,
    ],
)
def _emb_gather(emb_hbm, idx_hbm, out_hbm, idxv, rows, sem):
    wid = lax.axis_index("s") * 2 + lax.axis_index("c")

    def chunk(ci, carry):
        base = wid * RPW + ci * GC
        pltpu.sync_copy(idx_hbm.at[pl.ds(base, GC)], idxv)
        pltpu.async_copy(emb_hbm.at[idxv], rows, sem).wait()
        pltpu.sync_copy(rows, out_hbm.at[pl.ds(base, GC)])
        return carry

    lax.fori_loop(0, RPW // GC, chunk, 0)


def _make_edge_kernel(edge_fn, ts_w, td_w, acc_w):
    @functools.partial(
        pl.kernel,
        out_type=jax.ShapeDtypeStruct((2, NPAD, acc_w), jnp.float32),
        mesh=_mesh,
        compiler_params=pltpu.CompilerParams(use_tc_tiling_on_sc=False),
        scratch_types=[
            pltpu.VMEM((2, EC), jnp.int32),
            pltpu.VMEM((2, EC), jnp.int32),
            pltpu.VMEM((2, EC, ts_w), jnp.int32),
            pltpu.VMEM((2, EC, td_w), jnp.int32),
            pltpu.VMEM((EC, acc_w), jnp.float32),
            pltpu.VMEM_SHARED((NPAD, acc_w), jnp.float32),
            pltpu.SemaphoreType.DMA,
            pltpu.SemaphoreType.DMA,
            pltpu.SemaphoreType.DMA,
            pltpu.SemaphoreType.DMA,
        ],
    )
    def k(src_hbm, dst_hbm, tsrc_hbm, tdst_hbm, acc_hbm,
          sidx, didx, srows, drows, orows, accsh, ss0, ss1, sd0, sd1):
        cid = lax.axis_index("c")
        sid = lax.axis_index("s")
        sems_s = (ss0, ss1)
        sems_d = (sd0, sd1)
        epw0 = (EPAD - EPAD // 4) // 16
        epw1 = (EPAD // 4) // 16
        epw_c = jnp.where(cid == 0, epw0, epw1)
        off_c = jnp.where(cid == 0, 0, 16 * epw0)
        my_base = off_c + sid * epw_c
        nchunk_c = epw_c // EC

        zero16 = jnp.zeros((16,), jnp.float32)

        def zrow(i, c):
            for kk in range(acc_w // 16):
                orows[i, pl.ds(16 * kk, 16)] = zero16
            return c

        lax.fori_loop(0, EC, zrow, 0)
        rpw = NPAD // 16

        def zcp(kk, c):
            pltpu.sync_copy(orows, accsh.at[pl.ds(sid * rpw + kk * EC, EC)])
            return c

        lax.fori_loop(0, rpw // EC, zcp, 0)
        plsc.subcore_barrier()

        def issue(g, b):
            base = my_base + g * EC
            pltpu.sync_copy(src_hbm.at[pl.ds(base, EC)], sidx.at[b])
            pltpu.sync_copy(dst_hbm.at[pl.ds(base, EC)], didx.at[b])
            pltpu.async_copy(tsrc_hbm.at[sidx.at[b]], srows.at[b], sems_s[b])
            pltpu.async_copy(tdst_hbm.at[didx.at[b]], drows.at[b], sems_d[b])

        issue(0, 0)

        def outer(o, c):
            for b in range(2):
                g = 2 * o + b
                nb = 1 - b

                @pl.when(g + 1 < nchunk_c)
                def _():
                    issue(g + 1, nb)

                pltpu.make_async_copy(
                    tsrc_hbm.at[sidx.at[b]], srows.at[b], sems_s[b]).wait()
                pltpu.make_async_copy(
                    tdst_hbm.at[didx.at[b]], drows.at[b], sems_d[b]).wait()
                edge_fn(srows.at[b], drows.at[b], orows)
                pltpu.sync_copy(orows, accsh.at[didx.at[b]], add=True)
            return c

        lax.fori_loop(0, nchunk_c // 2, outer, 0)
        plsc.subcore_barrier()

        @pl.when(sid == 0)
        def _():
            pltpu.sync_copy(accsh, acc_hbm.at[cid])

    return k


def _lo16(v):
    return jax.lax.bitcast_convert_type(jnp.left_shift(v, 16), jnp.float32)


def _hi16(v):
    mask = jnp.full((16,), -65536, jnp.int32)
    return jax.lax.bitcast_convert_type(jnp.bitwise_and(v, mask), jnp.float32)


def _edge1_fn(srows, drows, orows):
    for i in range(EC):
        asrc = _lo16(srows[i, pl.ds(32, 16)])
        adst = _lo16(drows[i, pl.ds(0, 16)])
        al = asrc + adst
        al = jnp.where(al >= 0.0, al, 0.2 * al)
        e = jnp.exp(al)
        ha = srows[i, pl.ds(0, 16)]
        hb = srows[i, pl.ds(16, 16)]
        orows[i, pl.ds(0, 16)] = _lo16(ha) * e
        orows[i, pl.ds(16, 16)] = _hi16(ha) * e
        orows[i, pl.ds(32, 16)] = _lo16(hb) * e
        orows[i, pl.ds(48, 16)] = _hi16(hb) * e
        orows[i, pl.ds(64, 16)] = e


_edge1 = _make_edge_kernel(_edge1_fn, 48, 16, 80)


def _edge2_fn(srows, drows, orows):
    for i in range(EC):
        s0 = srows[i, pl.ds(0, 16)]
        asrc = _hi16(s0)
        adst = _lo16(drows[i, pl.ds(0, 16)])
        al = asrc + adst
        al = jnp.where(al >= 0.0, al, 0.2 * al)
        e = jnp.exp(al)
        orows[i, pl.ds(0, 16)] = _lo16(s0) * e
        orows[i, pl.ds(16, 16)] = e


_edge2 = _make_edge_kernel(_edge2_fn, 16, 16, 32)


def _rne16(x):
    xb = jax.lax.bitcast_convert_type(x, jnp.int32)
    r = jnp.right_shift(xb + 0x7FFF + jnp.bitwise_and(
        jnp.right_shift(xb, 16), 1), 16)
    return jnp.bitwise_and(r, 0xFFFF)


def _pack16(a, b):
    return jnp.bitwise_or(_rne16(a), jnp.left_shift(_rne16(b), 16))


def _tab1_body(h0_ref, w1p_ref, a2s_ref, a2d_ref, tsrc_ref, tdst_ref):
    h1p = jnp.dot(h0_ref[...], w1p_ref[...], preferred_element_type=jnp.float32)
    asrc2 = jnp.dot(h1p, a2s_ref[...], preferred_element_type=jnp.float32)
    adst2 = jnp.dot(h1p, a2d_ref[...], preferred_element_type=jnp.float32)
    zeros16 = jnp.zeros((h1p.shape[0], 16), jnp.float32)
    a = jnp.concatenate([h1p[:, 0:16], h1p[:, 32:48], asrc2[:, 0:16]], axis=1)
    b = jnp.concatenate([h1p[:, 16:32], h1p[:, 48:64], adst2[:, 0:16]], axis=1)
    tsrc_ref[...] = _pack16(a, b)
    tdst_ref[...] = _pack16(adst2, zeros16)


def _tab1(h0, W1p, A2s, A2d):
    bm = 512
    return pl.pallas_call(
        _tab1_body,
        out_shape=(
            jax.ShapeDtypeStruct((NPAD, 48), jnp.int32),
            jax.ShapeDtypeStruct((NPAD, 16), jnp.int32),
        ),
        grid=(NPAD // bm,),
        in_specs=[
            pl.BlockSpec((bm, D), lambda i: (i, 0)),
            pl.BlockSpec((D, 64), lambda i: (0, 0)),
            pl.BlockSpec((64, 16), lambda i: (0, 0)),
            pl.BlockSpec((64, 16), lambda i: (0, 0)),
        ],
        out_specs=(
            pl.BlockSpec((bm, 48), lambda i: (i, 0)),
            pl.BlockSpec((bm, 16), lambda i: (i, 0)),
        ),
    )(h0, W1p, A2s, A2d)


def _tab2_body(a0_ref, a1_ref, r8p_ref, pmt_ref, b1_ref, w2_ref,
               a2s_ref, a2d_ref, t2s_ref, t2d_ref):
    acc = a0_ref[...] + a1_ref[...]
    msgp = acc[:, :64]
    den = acc[:, 64:72]
    denp = jnp.dot(den, r8p_ref[...], preferred_element_type=jnp.float32)
    out1p = msgp / (denp + 1e-16)
    out1 = jnp.dot(out1p, pmt_ref[...], preferred_element_type=jnp.float32)
    z = out1 + b1_ref[0:1, :]
    h2 = jnp.where(z > 0.0, z, jnp.exp(z) - 1.0)
    g2 = jnp.dot(h2, w2_ref[...], preferred_element_type=jnp.float32)
    s2 = jnp.dot(g2, a2s_ref[...], preferred_element_type=jnp.float32)
    d2 = jnp.dot(g2, a2d_ref[...], preferred_element_type=jnp.float32)
    zeros16 = jnp.zeros((g2.shape[0], 16), jnp.float32)
    t2s_ref[...] = _pack16(g2, s2)
    t2d_ref[...] = _pack16(d2, zeros16)


def _tab2(acc0, acc1, R8p, PmT, b1m, W2, A2s2, A2d2):
    bm = 512
    return pl.pallas_call(
        _tab2_body,
        out_shape=(
            jax.ShapeDtypeStruct((NPAD, 16), jnp.int32),
            jax.ShapeDtypeStruct((NPAD, 16), jnp.int32),
        ),
        grid=(NPAD // bm,),
        in_specs=[
            pl.BlockSpec((bm, 80), lambda i: (i, 0)),
            pl.BlockSpec((bm, 80), lambda i: (i, 0)),
            pl.BlockSpec((8, 64), lambda i: (0, 0)),
            pl.BlockSpec((64, 64), lambda i: (0, 0)),
            pl.BlockSpec((8, 64), lambda i: (0, 0)),
            pl.BlockSpec((64, 16), lambda i: (0, 0)),
            pl.BlockSpec((16, 16), lambda i: (0, 0)),
            pl.BlockSpec((16, 16), lambda i: (0, 0)),
        ],
        out_specs=(
            pl.BlockSpec((bm, 16), lambda i: (i, 0)),
            pl.BlockSpec((bm, 16), lambda i: (i, 0)),
        ),
    )(acc0, acc1, R8p, PmT, b1m, W2, A2s2, A2d2)


def _fin_body(a0_ref, a1_ref, b2_ref, out_ref):
    acc = a0_ref[...] + a1_ref[...]
    msg = acc[:, :NC_OUT]
    den = acc[:, NC_OUT:2 * NC_OUT]
    out = msg / (den + 1e-16) + b2_ref[0:1, :]
    m = jnp.max(out, axis=-1, keepdims=True)
    s = out - m
    lse = jnp.log(jnp.sum(jnp.exp(s), axis=-1, keepdims=True))
    out_ref[...] = s - lse


def _finalize(acc0, acc1, b2m):
    bm = 512
    return pl.pallas_call(
        _fin_body,
        out_shape=jax.ShapeDtypeStruct((NPAD, NC_OUT), jnp.float32),
        grid=(NPAD // bm,),
        in_specs=[
            pl.BlockSpec((bm, 2 * NC_OUT), lambda i: (i, 0)),
            pl.BlockSpec((bm, 2 * NC_OUT), lambda i: (i, 0)),
            pl.BlockSpec((8, NC_OUT), lambda i: (0, 0)),
        ],
        out_specs=pl.BlockSpec((bm, NC_OUT), lambda i: (i, 0)),
    )(acc0, acc1, b2m)


def kernel(x, edge_index, emb, W1, a_src1, a_dst1, b1, W2, a_src2, a_dst2, b2):
    f32 = jnp.float32

    j = np.arange(64)
    prime_of_orig = (j % 8) * 8 + (j // 8)
    Pm = np.zeros((64, 64), np.float32)
    Pm[j, prime_of_orig] = 1.0
    Pm = jnp.asarray(Pm)
    W1p = W1 @ Pm
    rows = (np.arange(64) % 8) * 8 + (np.arange(64) // 8)
    h_idx = np.arange(64) // 8
    k_idx = np.arange(64) % 8
    Aps = jnp.zeros((64, 8), f32).at[rows, h_idx].set(a_src1[h_idx, k_idx])
    Apd = jnp.zeros((64, 8), f32).at[rows, h_idx].set(a_dst1[h_idx, k_idx])
    A2s = jnp.concatenate([Aps, Aps], axis=1)
    A2d = jnp.concatenate([Apd, Apd], axis=1)
    R8p = jnp.zeros((8, 64), f32).at[h_idx, rows].set(1.0)
    PmT = Pm.T
    b1m = jnp.broadcast_to(b1[None, :], (8, 64))
    A2s2 = jnp.broadcast_to(a_src2[0][:, None], (16, 16))
    A2d2 = jnp.broadcast_to(a_dst2[0][:, None], (16, 16))
    b2m = jnp.broadcast_to(b2[None, :], (8, NC_OUT))


    idx = jnp.pad(x[:, 0].astype(jnp.int32), (0, NPAD - N))
    srcp = jnp.pad(edge_index[0].astype(jnp.int32), (0, EPAD - E),
                   constant_values=N)
    dstp = jnp.pad(edge_index[1].astype(jnp.int32), (0, EPAD - E),
                   constant_values=N)
    h0 = _emb_gather(emb, idx)
    ts1, td1 = _tab1(h0, W1p, A2s, A2d)
    acc1 = _edge1(srcp, dstp, ts1, td1)
    t2s, t2d = _tab2(acc1[0], acc1[1], R8p, PmT, b1m, W2, A2s2, A2d2)
    acc2 = _edge2(srcp, dstp, t2s, t2d)
    out = _finalize(acc2[0], acc2[1], b2m)
    return out[:N]


# speedup vs baseline: 4.3553x; 1.7456x over previous
import functools
import jax
import jax.numpy as jnp
import numpy as np
from jax import lax
from jax.experimental import pallas as pl
from jax.experimental.pallas import tpu as pltpu
from jax.experimental.pallas import tpu_sc as plsc

N = 10000
E = 320000
D = 128
HEADS = 8
HID = 8
NC_OUT = 16

NPAD = 10240
EPAD = 327680
EPW = EPAD // 32
EC = 128
NCHUNK = EPW // EC
GC = 64
RPW = NPAD // 32

_mesh = plsc.VectorSubcoreMesh(core_axis_name="c", subcore_axis_name="s")


@functools.partial(
    pl.kernel,
    out_type=jax.ShapeDtypeStruct((NPAD, D), jnp.float32),
    mesh=_mesh,
    scratch_types=[
        pltpu.VMEM((GC,), jnp.int32),
        pltpu.VMEM((GC, D), jnp.float32),
        pltpu.SemaphoreType.DMA,
    ],
)
def _emb_gather(emb_hbm, idx_hbm, out_hbm, idxv, rows, sem):
    wid = lax.axis_index("s") * 2 + lax.axis_index("c")

    def chunk(ci, carry):
        base = wid * RPW + ci * GC
        pltpu.sync_copy(idx_hbm.at[pl.ds(base, GC)], idxv)
        pltpu.async_copy(emb_hbm.at[idxv], rows, sem).wait()
        pltpu.sync_copy(rows, out_hbm.at[pl.ds(base, GC)])
        return carry

    lax.fori_loop(0, RPW // GC, chunk, 0)


def _make_edge_kernel(edge_fn, ts_w, td_w, acc_w):
    epw0 = (EPAD - EPAD // 4) // 16
    epw1 = (EPAD // 4) // 16
    nch0 = epw0 // EC
    nch1 = epw1 // EC

    @functools.partial(
        pl.kernel,
        out_type=jax.ShapeDtypeStruct((2, NPAD, acc_w), jnp.float32),
        mesh=_mesh,
        compiler_params=pltpu.CompilerParams(use_tc_tiling_on_sc=False),
        scratch_types=[
            pltpu.VMEM((nch0, EC), jnp.int32),
            pltpu.VMEM((nch0, EC), jnp.int32),
            pltpu.VMEM((2, EC, ts_w), jnp.int32),
            pltpu.VMEM((2, EC, td_w), jnp.int32),
            pltpu.VMEM((EC, acc_w), jnp.float32),
            pltpu.VMEM_SHARED((NPAD, acc_w), jnp.float32),
            pltpu.SemaphoreType.DMA,
            pltpu.SemaphoreType.DMA,
            pltpu.SemaphoreType.DMA,
            pltpu.SemaphoreType.DMA,
        ],
    )
    def k(src_hbm, dst_hbm, tsrc_hbm, tdst_hbm, acc_hbm,
          sidx, didx, srows, drows, orows, accsh, ss0, ss1, sd0, sd1):
        cid = lax.axis_index("c")
        sid = lax.axis_index("s")
        sems_s = (ss0, ss1)
        sems_d = (sd0, sd1)
        epw_c = jnp.where(cid == 0, epw0, epw1)
        off_c = jnp.where(cid == 0, 0, 16 * epw0)
        my_base = off_c + sid * epw_c
        nchunk_c = epw_c // EC
        row0 = my_base // EC

        @pl.when(cid == 0)
        def _():
            pltpu.sync_copy(src_hbm.at[pl.ds(row0, nch0)],
                            sidx.at[pl.ds(0, nch0)])
            pltpu.sync_copy(dst_hbm.at[pl.ds(row0, nch0)],
                            didx.at[pl.ds(0, nch0)])

        @pl.when(cid == 1)
        def _():
            pltpu.sync_copy(src_hbm.at[pl.ds(row0, nch1)],
                            sidx.at[pl.ds(0, nch1)])
            pltpu.sync_copy(dst_hbm.at[pl.ds(row0, nch1)],
                            didx.at[pl.ds(0, nch1)])

        zero16 = jnp.zeros((16,), jnp.float32)

        def zrow(i, c):
            for kk in range(acc_w // 16):
                orows[i, pl.ds(16 * kk, 16)] = zero16
            return c

        lax.fori_loop(0, EC, zrow, 0)
        rpw = NPAD // 16

        def zcp(kk, c):
            pltpu.sync_copy(orows, accsh.at[pl.ds(sid * rpw + kk * EC, EC)])
            return c

        lax.fori_loop(0, rpw // EC, zcp, 0)
        plsc.subcore_barrier()

        def issue(g, b):
            pltpu.async_copy(tsrc_hbm.at[sidx.at[g]], srows.at[b], sems_s[b])
            pltpu.async_copy(tdst_hbm.at[didx.at[g]], drows.at[b], sems_d[b])

        issue(0, 0)

        def outer(o, c):
            for b in range(2):
                g = 2 * o + b
                nb = 1 - b

                @pl.when(g + 1 < nchunk_c)
                def _():
                    issue(g + 1, nb)

                pltpu.make_async_copy(
                    tsrc_hbm.at[sidx.at[g]], srows.at[b], sems_s[b]).wait()
                pltpu.make_async_copy(
                    tdst_hbm.at[didx.at[g]], drows.at[b], sems_d[b]).wait()
                edge_fn(srows.at[b], drows.at[b], orows)
                pltpu.sync_copy(orows, accsh.at[didx.at[g]], add=True)
            return c

        lax.fori_loop(0, nchunk_c // 2, outer, 0)
        plsc.subcore_barrier()

        @pl.when(sid == 0)
        def _():
            pltpu.sync_copy(accsh, acc_hbm.at[cid])

    return k


def _lo16(v):
    return jax.lax.bitcast_convert_type(jnp.left_shift(v, 16), jnp.float32)


def _hi16(v):
    mask = jnp.full((16,), -65536, jnp.int32)
    return jax.lax.bitcast_convert_type(jnp.bitwise_and(v, mask), jnp.float32)


def _edge1_fn(srows, drows, orows):
    for i in range(EC):
        asrc = _lo16(srows[i, pl.ds(32, 16)])
        adst = _lo16(drows[i, pl.ds(0, 16)])
        al = asrc + adst
        al = jnp.where(al >= 0.0, al, 0.2 * al)
        e = jnp.exp(al)
        ha = srows[i, pl.ds(0, 16)]
        hb = srows[i, pl.ds(16, 16)]
        orows[i, pl.ds(0, 16)] = _lo16(ha) * e
        orows[i, pl.ds(16, 16)] = _hi16(ha) * e
        orows[i, pl.ds(32, 16)] = _lo16(hb) * e
        orows[i, pl.ds(48, 16)] = _hi16(hb) * e
        orows[i, pl.ds(64, 16)] = e


_edge1 = _make_edge_kernel(_edge1_fn, 48, 16, 80)


def _edge2_fn(srows, drows, orows):
    for i in range(EC):
        s0 = srows[i, pl.ds(0, 16)]
        asrc = _hi16(s0)
        adst = _lo16(drows[i, pl.ds(0, 16)])
        al = asrc + adst
        al = jnp.where(al >= 0.0, al, 0.2 * al)
        e = jnp.exp(al)
        orows[i, pl.ds(0, 16)] = _lo16(s0) * e
        orows[i, pl.ds(16, 16)] = e


_edge2 = _make_edge_kernel(_edge2_fn, 16, 16, 32)


def _rne16(x):
    xb = jax.lax.bitcast_convert_type(x, jnp.int32)
    r = jnp.right_shift(xb + 0x7FFF + jnp.bitwise_and(
        jnp.right_shift(xb, 16), 1), 16)
    return jnp.bitwise_and(r, 0xFFFF)


def _pack16(a, b):
    return jnp.bitwise_or(_rne16(a), jnp.left_shift(_rne16(b), 16))


def _tab1_body(h0_ref, w1p_ref, a2s_ref, a2d_ref, tsrc_ref, tdst_ref):
    h1p = jnp.dot(h0_ref[...], w1p_ref[...], preferred_element_type=jnp.float32)
    asrc2 = jnp.dot(h1p, a2s_ref[...], preferred_element_type=jnp.float32)
    adst2 = jnp.dot(h1p, a2d_ref[...], preferred_element_type=jnp.float32)
    zeros16 = jnp.zeros((h1p.shape[0], 16), jnp.float32)
    a = jnp.concatenate([h1p[:, 0:16], h1p[:, 32:48], asrc2[:, 0:16]], axis=1)
    b = jnp.concatenate([h1p[:, 16:32], h1p[:, 48:64], adst2[:, 0:16]], axis=1)
    tsrc_ref[...] = _pack16(a, b)
    tdst_ref[...] = _pack16(adst2, zeros16)


def _tab1(h0, W1p, A2s, A2d):
    bm = 512
    return pl.pallas_call(
        _tab1_body,
        out_shape=(
            jax.ShapeDtypeStruct((NPAD, 48), jnp.int32),
            jax.ShapeDtypeStruct((NPAD, 16), jnp.int32),
        ),
        grid=(NPAD // bm,),
        in_specs=[
            pl.BlockSpec((bm, D), lambda i: (i, 0)),
            pl.BlockSpec((D, 64), lambda i: (0, 0)),
            pl.BlockSpec((64, 16), lambda i: (0, 0)),
            pl.BlockSpec((64, 16), lambda i: (0, 0)),
        ],
        out_specs=(
            pl.BlockSpec((bm, 48), lambda i: (i, 0)),
            pl.BlockSpec((bm, 16), lambda i: (i, 0)),
        ),
    )(h0, W1p, A2s, A2d)


def _tab2_body(a0_ref, a1_ref, r8p_ref, pmt_ref, b1_ref, w2_ref,
               a2s_ref, a2d_ref, t2s_ref, t2d_ref):
    acc = a0_ref[...] + a1_ref[...]
    msgp = acc[:, :64]
    den = acc[:, 64:72]
    denp = jnp.dot(den, r8p_ref[...], preferred_element_type=jnp.float32)
    out1p = msgp / (denp + 1e-16)
    out1 = jnp.dot(out1p, pmt_ref[...], preferred_element_type=jnp.float32)
    z = out1 + b1_ref[0:1, :]
    h2 = jnp.where(z > 0.0, z, jnp.exp(z) - 1.0)
    g2 = jnp.dot(h2, w2_ref[...], preferred_element_type=jnp.float32)
    s2 = jnp.dot(g2, a2s_ref[...], preferred_element_type=jnp.float32)
    d2 = jnp.dot(g2, a2d_ref[...], preferred_element_type=jnp.float32)
    zeros16 = jnp.zeros((g2.shape[0], 16), jnp.float32)
    t2s_ref[...] = _pack16(g2, s2)
    t2d_ref[...] = _pack16(d2, zeros16)


def _tab2(acc0, acc1, R8p, PmT, b1m, W2, A2s2, A2d2):
    bm = 512
    return pl.pallas_call(
        _tab2_body,
        out_shape=(
            jax.ShapeDtypeStruct((NPAD, 16), jnp.int32),
            jax.ShapeDtypeStruct((NPAD, 16), jnp.int32),
        ),
        grid=(NPAD // bm,),
        in_specs=[
            pl.BlockSpec((bm, 80), lambda i: (i, 0)),
            pl.BlockSpec((bm, 80), lambda i: (i, 0)),
            pl.BlockSpec((8, 64), lambda i: (0, 0)),
            pl.BlockSpec((64, 64), lambda i: (0, 0)),
            pl.BlockSpec((8, 64), lambda i: (0, 0)),
            pl.BlockSpec((64, 16), lambda i: (0, 0)),
            pl.BlockSpec((16, 16), lambda i: (0, 0)),
            pl.BlockSpec((16, 16), lambda i: (0, 0)),
        ],
        out_specs=(
            pl.BlockSpec((bm, 16), lambda i: (i, 0)),
            pl.BlockSpec((bm, 16), lambda i: (i, 0)),
        ),
    )(acc0, acc1, R8p, PmT, b1m, W2, A2s2, A2d2)


def _fin_body(a0_ref, a1_ref, b2_ref, out_ref):
    acc = a0_ref[...] + a1_ref[...]
    msg = acc[:, :NC_OUT]
    den = acc[:, NC_OUT:2 * NC_OUT]
    out = msg / (den + 1e-16) + b2_ref[0:1, :]
    m = jnp.max(out, axis=-1, keepdims=True)
    s = out - m
    lse = jnp.log(jnp.sum(jnp.exp(s), axis=-1, keepdims=True))
    out_ref[...] = s - lse


def _finalize(acc0, acc1, b2m):
    bm = 512
    return pl.pallas_call(
        _fin_body,
        out_shape=jax.ShapeDtypeStruct((NPAD, NC_OUT), jnp.float32),
        grid=(NPAD // bm,),
        in_specs=[
            pl.BlockSpec((bm, 2 * NC_OUT), lambda i: (i, 0)),
            pl.BlockSpec((bm, 2 * NC_OUT), lambda i: (i, 0)),
            pl.BlockSpec((8, NC_OUT), lambda i: (0, 0)),
        ],
        out_specs=pl.BlockSpec((bm, NC_OUT), lambda i: (i, 0)),
    )(acc0, acc1, b2m)


def kernel(x, edge_index, emb, W1, a_src1, a_dst1, b1, W2, a_src2, a_dst2, b2):
    f32 = jnp.float32

    j = np.arange(64)
    prime_of_orig = (j % 8) * 8 + (j // 8)
    Pm = np.zeros((64, 64), np.float32)
    Pm[j, prime_of_orig] = 1.0
    Pm = jnp.asarray(Pm)
    W1p = W1 @ Pm
    rows = (np.arange(64) % 8) * 8 + (np.arange(64) // 8)
    h_idx = np.arange(64) // 8
    k_idx = np.arange(64) % 8
    Aps = jnp.zeros((64, 8), f32).at[rows, h_idx].set(a_src1[h_idx, k_idx])
    Apd = jnp.zeros((64, 8), f32).at[rows, h_idx].set(a_dst1[h_idx, k_idx])
    A2s = jnp.concatenate([Aps, Aps], axis=1)
    A2d = jnp.concatenate([Apd, Apd], axis=1)
    R8p = jnp.zeros((8, 64), f32).at[h_idx, rows].set(1.0)
    PmT = Pm.T
    b1m = jnp.broadcast_to(b1[None, :], (8, 64))
    A2s2 = jnp.broadcast_to(a_src2[0][:, None], (16, 16))
    A2d2 = jnp.broadcast_to(a_dst2[0][:, None], (16, 16))
    b2m = jnp.broadcast_to(b2[None, :], (8, NC_OUT))


    idx = jnp.pad(x[:, 0].astype(jnp.int32), (0, NPAD - N))
    srcp = jnp.pad(edge_index[0].astype(jnp.int32), (0, EPAD - E),
                   constant_values=N).reshape(EPAD // EC, EC)
    dstp = jnp.pad(edge_index[1].astype(jnp.int32), (0, EPAD - E),
                   constant_values=N).reshape(EPAD // EC, EC)
    h0 = _emb_gather(emb, idx)
    ts1, td1 = _tab1(h0, W1p, A2s, A2d)
    acc1 = _edge1(srcp, dstp, ts1, td1)
    t2s, t2d = _tab2(acc1[0], acc1[1], R8p, PmT, b1m, W2, A2s2, A2d2)
    acc2 = _edge2(srcp, dstp, t2s, t2d)
    out = _finalize(acc2[0], acc2[1], b2m)
    return out[:N]


# speedup vs baseline: 4.6848x; 1.0756x over previous
import functools
import jax
import jax.numpy as jnp
import numpy as np
from jax import lax
from jax.experimental import pallas as pl
from jax.experimental.pallas import tpu as pltpu
from jax.experimental.pallas import tpu_sc as plsc

N = 10000
E = 320000
D = 128
HEADS = 8
HID = 8
NC_OUT = 16

NPAD = 10240
EPAD = 327680
EPW = EPAD // 32
EC = 128
NCHUNK = EPW // EC
GC = 64
RPW = NPAD // 32

_mesh = plsc.VectorSubcoreMesh(core_axis_name="c", subcore_axis_name="s")


@functools.partial(
    pl.kernel,
    out_type=jax.ShapeDtypeStruct((NPAD, D), jnp.float32),
    mesh=_mesh,
    scratch_types=[
        pltpu.VMEM((GC,), jnp.int32),
        pltpu.VMEM((GC, D), jnp.float32),
        pltpu.SemaphoreType.DMA,
    ],
)
def _emb_gather(emb_hbm, idx_hbm, out_hbm, idxv, rows, sem):
    wid = lax.axis_index("s") * 2 + lax.axis_index("c")

    def chunk(ci, carry):
        base = wid * RPW + ci * GC
        pltpu.sync_copy(idx_hbm.at[pl.ds(base, GC)], idxv)
        pltpu.async_copy(emb_hbm.at[idxv], rows, sem).wait()
        pltpu.sync_copy(rows, out_hbm.at[pl.ds(base, GC)])
        return carry

    lax.fori_loop(0, RPW // GC, chunk, 0)


def _make_edge_kernel(edge_fn, ts_w, td_w, acc_w):
    epw0 = (EPAD - EPAD // 4) // 16
    epw1 = (EPAD // 4) // 16
    nch0 = epw0 // EC
    nch1 = epw1 // EC

    @functools.partial(
        pl.kernel,
        out_type=jax.ShapeDtypeStruct((2, NPAD, acc_w), jnp.float32),
        mesh=_mesh,
        compiler_params=pltpu.CompilerParams(use_tc_tiling_on_sc=False),
        scratch_types=[
            pltpu.VMEM((nch0, EC), jnp.int32),
            pltpu.VMEM((nch0, EC), jnp.int32),
            pltpu.VMEM((2, EC, ts_w), jnp.int32),
            pltpu.VMEM((2, EC, td_w), jnp.int32),
            pltpu.VMEM((2, EC, acc_w), jnp.float32),
            pltpu.VMEM_SHARED((NPAD, acc_w), jnp.float32),
            pltpu.SemaphoreType.DMA,
            pltpu.SemaphoreType.DMA,
            pltpu.SemaphoreType.DMA,
            pltpu.SemaphoreType.DMA,
            pltpu.SemaphoreType.DMA,
            pltpu.SemaphoreType.DMA,
        ],
    )
    def k(src_hbm, dst_hbm, tsrc_hbm, tdst_hbm, acc_hbm,
          sidx, didx, srows, drows, orows, accsh,
          ss0, ss1, sd0, sd1, so0, so1):
        cid = lax.axis_index("c")
        sid = lax.axis_index("s")
        sems_s = (ss0, ss1)
        sems_d = (sd0, sd1)
        sems_o = (so0, so1)
        epw_c = jnp.where(cid == 0, epw0, epw1)
        off_c = jnp.where(cid == 0, 0, 16 * epw0)
        my_base = off_c + sid * epw_c
        nchunk_c = epw_c // EC
        row0 = my_base // EC

        @pl.when(cid == 0)
        def _():
            pltpu.sync_copy(src_hbm.at[pl.ds(row0, nch0)],
                            sidx.at[pl.ds(0, nch0)])
            pltpu.sync_copy(dst_hbm.at[pl.ds(row0, nch0)],
                            didx.at[pl.ds(0, nch0)])

        @pl.when(cid == 1)
        def _():
            pltpu.sync_copy(src_hbm.at[pl.ds(row0, nch1)],
                            sidx.at[pl.ds(0, nch1)])
            pltpu.sync_copy(dst_hbm.at[pl.ds(row0, nch1)],
                            didx.at[pl.ds(0, nch1)])

        zero16 = jnp.zeros((16,), jnp.float32)

        def zrow(i, c):
            for bb in range(2):
                for kk in range(acc_w // 16):
                    orows[bb, i, pl.ds(16 * kk, 16)] = zero16
            return c

        lax.fori_loop(0, EC, zrow, 0)
        rpw = NPAD // 16

        def zcp(kk, c):
            pltpu.sync_copy(orows.at[0],
                            accsh.at[pl.ds(sid * rpw + kk * EC, EC)])
            return c

        lax.fori_loop(0, rpw // EC, zcp, 0)
        plsc.subcore_barrier()

        def issue(g, b):
            pltpu.async_copy(tsrc_hbm.at[sidx.at[g]], srows.at[b], sems_s[b])
            pltpu.async_copy(tdst_hbm.at[didx.at[g]], drows.at[b], sems_d[b])

        issue(0, 0)

        def outer(o, c):
            for b in range(2):
                g = 2 * o + b
                nb = 1 - b

                @pl.when(g + 1 < nchunk_c)
                def _():
                    issue(g + 1, nb)

                pltpu.make_async_copy(
                    tsrc_hbm.at[sidx.at[g]], srows.at[b], sems_s[b]).wait()
                pltpu.make_async_copy(
                    tdst_hbm.at[didx.at[g]], drows.at[b], sems_d[b]).wait()

                @pl.when(g >= 2)
                def _():
                    pltpu.make_async_copy(
                        orows.at[b], accsh.at[didx.at[g]], sems_o[b]).wait()

                edge_fn(srows.at[b], drows.at[b], orows.at[b])
                pltpu.async_copy(orows.at[b], accsh.at[didx.at[g]],
                                 sems_o[b], add=True)
            return c

        lax.fori_loop(0, nchunk_c // 2, outer, 0)
        for b in range(2):
            pltpu.make_async_copy(
                orows.at[b], accsh.at[pl.ds(0, EC)], sems_o[b]).wait()
        plsc.subcore_barrier()

        @pl.when(sid == 0)
        def _():
            pltpu.sync_copy(accsh, acc_hbm.at[cid])

    return k


def _lo16(v):
    return jax.lax.bitcast_convert_type(jnp.left_shift(v, 16), jnp.float32)


def _hi16(v):
    mask = jnp.full((16,), -65536, jnp.int32)
    return jax.lax.bitcast_convert_type(jnp.bitwise_and(v, mask), jnp.float32)


def _edge1_fn(srows, drows, orows):
    for i in range(EC):
        asrc = _lo16(srows[i, pl.ds(32, 16)])
        adst = _lo16(drows[i, pl.ds(0, 16)])
        al = asrc + adst
        al = jnp.where(al >= 0.0, al, 0.2 * al)
        e = jnp.exp(al)
        ha = srows[i, pl.ds(0, 16)]
        hb = srows[i, pl.ds(16, 16)]
        orows[i, pl.ds(0, 16)] = _lo16(ha) * e
        orows[i, pl.ds(16, 16)] = _hi16(ha) * e
        orows[i, pl.ds(32, 16)] = _lo16(hb) * e
        orows[i, pl.ds(48, 16)] = _hi16(hb) * e
        orows[i, pl.ds(64, 16)] = e


_edge1 = _make_edge_kernel(_edge1_fn, 48, 16, 80)


def _edge2_fn(srows, drows, orows):
    for i in range(EC):
        s0 = srows[i, pl.ds(0, 16)]
        asrc = _hi16(s0)
        adst = _lo16(drows[i, pl.ds(0, 16)])
        al = asrc + adst
        al = jnp.where(al >= 0.0, al, 0.2 * al)
        e = jnp.exp(al)
        orows[i, pl.ds(0, 16)] = _lo16(s0) * e
        orows[i, pl.ds(16, 16)] = e


_edge2 = _make_edge_kernel(_edge2_fn, 16, 16, 32)


def _rne16(x):
    xb = jax.lax.bitcast_convert_type(x, jnp.int32)
    r = jnp.right_shift(xb + 0x7FFF + jnp.bitwise_and(
        jnp.right_shift(xb, 16), 1), 16)
    return jnp.bitwise_and(r, 0xFFFF)


def _pack16(a, b):
    return jnp.bitwise_or(_rne16(a), jnp.left_shift(_rne16(b), 16))


def _tab1_body(h0_ref, w1p_ref, a2s_ref, a2d_ref, tsrc_ref, tdst_ref):
    h1p = jnp.dot(h0_ref[...], w1p_ref[...], preferred_element_type=jnp.float32)
    asrc2 = jnp.dot(h1p, a2s_ref[...], preferred_element_type=jnp.float32)
    adst2 = jnp.dot(h1p, a2d_ref[...], preferred_element_type=jnp.float32)
    zeros16 = jnp.zeros((h1p.shape[0], 16), jnp.float32)
    a = jnp.concatenate([h1p[:, 0:16], h1p[:, 32:48], asrc2[:, 0:16]], axis=1)
    b = jnp.concatenate([h1p[:, 16:32], h1p[:, 48:64], adst2[:, 0:16]], axis=1)
    tsrc_ref[...] = _pack16(a, b)
    tdst_ref[...] = _pack16(adst2, zeros16)


def _tab1(h0, W1p, A2s, A2d):
    bm = 512
    return pl.pallas_call(
        _tab1_body,
        out_shape=(
            jax.ShapeDtypeStruct((NPAD, 48), jnp.int32),
            jax.ShapeDtypeStruct((NPAD, 16), jnp.int32),
        ),
        grid=(NPAD // bm,),
        in_specs=[
            pl.BlockSpec((bm, D), lambda i: (i, 0)),
            pl.BlockSpec((D, 64), lambda i: (0, 0)),
            pl.BlockSpec((64, 16), lambda i: (0, 0)),
            pl.BlockSpec((64, 16), lambda i: (0, 0)),
        ],
        out_specs=(
            pl.BlockSpec((bm, 48), lambda i: (i, 0)),
            pl.BlockSpec((bm, 16), lambda i: (i, 0)),
        ),
    )(h0, W1p, A2s, A2d)


def _tab2_body(a0_ref, a1_ref, r8p_ref, pmt_ref, b1_ref, w2_ref,
               a2s_ref, a2d_ref, t2s_ref, t2d_ref):
    acc = a0_ref[...] + a1_ref[...]
    msgp = acc[:, :64]
    den = acc[:, 64:72]
    denp = jnp.dot(den, r8p_ref[...], preferred_element_type=jnp.float32)
    out1p = msgp / (denp + 1e-16)
    out1 = jnp.dot(out1p, pmt_ref[...], preferred_element_type=jnp.float32)
    z = out1 + b1_ref[0:1, :]
    h2 = jnp.where(z > 0.0, z, jnp.exp(z) - 1.0)
    g2 = jnp.dot(h2, w2_ref[...], preferred_element_type=jnp.float32)
    s2 = jnp.dot(g2, a2s_ref[...], preferred_element_type=jnp.float32)
    d2 = jnp.dot(g2, a2d_ref[...], preferred_element_type=jnp.float32)
    zeros16 = jnp.zeros((g2.shape[0], 16), jnp.float32)
    t2s_ref[...] = _pack16(g2, s2)
    t2d_ref[...] = _pack16(d2, zeros16)


def _tab2(acc0, acc1, R8p, PmT, b1m, W2, A2s2, A2d2):
    bm = 512
    return pl.pallas_call(
        _tab2_body,
        out_shape=(
            jax.ShapeDtypeStruct((NPAD, 16), jnp.int32),
            jax.ShapeDtypeStruct((NPAD, 16), jnp.int32),
        ),
        grid=(NPAD // bm,),
        in_specs=[
            pl.BlockSpec((bm, 80), lambda i: (i, 0)),
            pl.BlockSpec((bm, 80), lambda i: (i, 0)),
            pl.BlockSpec((8, 64), lambda i: (0, 0)),
            pl.BlockSpec((64, 64), lambda i: (0, 0)),
            pl.BlockSpec((8, 64), lambda i: (0, 0)),
            pl.BlockSpec((64, 16), lambda i: (0, 0)),
            pl.BlockSpec((16, 16), lambda i: (0, 0)),
            pl.BlockSpec((16, 16), lambda i: (0, 0)),
        ],
        out_specs=(
            pl.BlockSpec((bm, 16), lambda i: (i, 0)),
            pl.BlockSpec((bm, 16), lambda i: (i, 0)),
        ),
    )(acc0, acc1, R8p, PmT, b1m, W2, A2s2, A2d2)


def _fin_body(a0_ref, a1_ref, b2_ref, out_ref):
    acc = a0_ref[...] + a1_ref[...]
    msg = acc[:, :NC_OUT]
    den = acc[:, NC_OUT:2 * NC_OUT]
    out = msg / (den + 1e-16) + b2_ref[0:1, :]
    m = jnp.max(out, axis=-1, keepdims=True)
    s = out - m
    lse = jnp.log(jnp.sum(jnp.exp(s), axis=-1, keepdims=True))
    out_ref[...] = s - lse


def _finalize(acc0, acc1, b2m):
    bm = 512
    return pl.pallas_call(
        _fin_body,
        out_shape=jax.ShapeDtypeStruct((NPAD, NC_OUT), jnp.float32),
        grid=(NPAD // bm,),
        in_specs=[
            pl.BlockSpec((bm, 2 * NC_OUT), lambda i: (i, 0)),
            pl.BlockSpec((bm, 2 * NC_OUT), lambda i: (i, 0)),
            pl.BlockSpec((8, NC_OUT), lambda i: (0, 0)),
        ],
        out_specs=pl.BlockSpec((bm, NC_OUT), lambda i: (i, 0)),
    )(acc0, acc1, b2m)


def kernel(x, edge_index, emb, W1, a_src1, a_dst1, b1, W2, a_src2, a_dst2, b2):
    f32 = jnp.float32

    j = np.arange(64)
    prime_of_orig = (j % 8) * 8 + (j // 8)
    Pm = np.zeros((64, 64), np.float32)
    Pm[j, prime_of_orig] = 1.0
    Pm = jnp.asarray(Pm)
    W1p = W1 @ Pm
    rows = (np.arange(64) % 8) * 8 + (np.arange(64) // 8)
    h_idx = np.arange(64) // 8
    k_idx = np.arange(64) % 8
    Aps = jnp.zeros((64, 8), f32).at[rows, h_idx].set(a_src1[h_idx, k_idx])
    Apd = jnp.zeros((64, 8), f32).at[rows, h_idx].set(a_dst1[h_idx, k_idx])
    A2s = jnp.concatenate([Aps, Aps], axis=1)
    A2d = jnp.concatenate([Apd, Apd], axis=1)
    R8p = jnp.zeros((8, 64), f32).at[h_idx, rows].set(1.0)
    PmT = Pm.T
    b1m = jnp.broadcast_to(b1[None, :], (8, 64))
    A2s2 = jnp.broadcast_to(a_src2[0][:, None], (16, 16))
    A2d2 = jnp.broadcast_to(a_dst2[0][:, None], (16, 16))
    b2m = jnp.broadcast_to(b2[None, :], (8, NC_OUT))


    idx = jnp.pad(x[:, 0].astype(jnp.int32), (0, NPAD - N))
    srcp = jnp.pad(edge_index[0].astype(jnp.int32), (0, EPAD - E),
                   constant_values=N).reshape(EPAD // EC, EC)
    dstp = jnp.pad(edge_index[1].astype(jnp.int32), (0, EPAD - E),
                   constant_values=N).reshape(EPAD // EC, EC)
    h0 = _emb_gather(emb, idx)
    ts1, td1 = _tab1(h0, W1p, A2s, A2d)
    acc1 = _edge1(srcp, dstp, ts1, td1)
    t2s, t2d = _tab2(acc1[0], acc1[1], R8p, PmT, b1m, W2, A2s2, A2d2)
    acc2 = _edge2(srcp, dstp, t2s, t2d)
    out = _finalize(acc2[0], acc2[1], b2m)
    return out[:N]
